# baseline (speedup 1.0000x reference)
import jax
import jax.numpy as jnp
from jax import lax
from jax.experimental import pallas as pl
from jax.experimental.pallas import tpu as pltpu
from jax.experimental.pallas import tpu_sc as plsc

_N = 10000
_E = 320000
_D = 128
_H = 128
_C = 10
_GF = 32
_G = 64
_EPS = 1e-5

_NC = 2
_NS = 16
_NW = _NC * _NS
_K = 64
_NCHUNK = 160
_PCH = 40
_NPH = 4
_EPAD = _NW * _NCHUNK * _K
_AROWS = 10240
_RPS = _AROWS // _NS
_BLK = 256
_NBLK = _AROWS // _BLK

_mesh = plsc.VectorSubcoreMesh(core_axis_name="c", subcore_axis_name="s")


def _deg_body(dst_hbm, ones_hbm, zeros_hbm, out_hbm, idx_d, ones_v, acc,
              sem_s):
    cid = lax.axis_index("c")
    sid = lax.axis_index("s")
    wid = sid * _NC + cid
    pltpu.sync_copy(zeros_hbm, acc.at[pl.ds(sid * _RPS, _RPS)])
    pltpu.sync_copy(ones_hbm, ones_v)
    pltpu.sync_copy(dst_hbm.at[wid], idx_d)
    plsc.subcore_barrier()

    def s_start(j, b):
        pltpu.async_copy(ones_v, acc.at[idx_d.at[j]], sem_s.at[b], add=True)

    def s_wait(j, b):
        pltpu.make_async_copy(ones_v, acc.at[idx_d.at[j]],
                              sem_s.at[b]).wait()

    def body(j, carry):
        @pl.when(j >= _NBUF)
        def _():
            s_wait(j - _NBUF, 0)
        s_start(j, 0)
        return carry

    lax.fori_loop(0, _NCHUNK, body, 0)
    for m in range(_NCHUNK - _NBUF, _NCHUNK):
        s_wait(m, 0)
    plsc.subcore_barrier()
    pltpu.sync_copy(acc.at[pl.ds(sid * _RPS, _RPS)],
                    out_hbm.at[cid, pl.ds(sid * _RPS, _RPS)])


_deg = pl.kernel(
    _deg_body,
    out_type=jax.ShapeDtypeStruct((_NC, _AROWS, _D), jnp.float32),
    mesh=_mesh,
    scratch_types=[
        pltpu.VMEM((_NCHUNK, _K), jnp.int32),
        pltpu.VMEM((_K, _D), jnp.float32),
        pltpu.VMEM_SHARED((_AROWS, _D), jnp.float32),
        pltpu.SemaphoreType.DMA((1,)),
    ],
)


_NBUF = 4
_PCH2 = 64
_C0 = 256
_C1 = 64


def _route_body(x_hbm, src_hbm, dst_hbm, zeros_hbm, out_hbm,
                idx_s, idx_d, rows, acc, sem_g, sem_s):
    cid = lax.axis_index("c")
    sid = lax.axis_index("s")
    pltpu.sync_copy(zeros_hbm, acc.at[pl.ds(sid * _RPS, _RPS)])
    plsc.subcore_barrier()

    def g_start(j, b):
        pltpu.async_copy(x_hbm.at[idx_s.at[j]], rows.at[b], sem_g.at[b])

    def g_wait(j, b):
        pltpu.make_async_copy(x_hbm.at[idx_s.at[j]], rows.at[b],
                              sem_g.at[b]).wait()

    def s_start(j, b):
        pltpu.async_copy(rows.at[b], acc.at[idx_d.at[j]], sem_s.at[b],
                         add=True)

    def s_wait(j, b):
        pltpu.make_async_copy(rows.at[b], acc.at[idx_d.at[j]],
                              sem_s.at[b]).wait()

    def run_core(count, base):
        start0 = base + sid * count
        ngrp = _PCH2 // _NBUF

        def phase(p, carry):
            off = start0 + p * _PCH2
            pltpu.sync_copy(src_hbm.at[pl.ds(off, _PCH2)], idx_s)
            pltpu.sync_copy(dst_hbm.at[pl.ds(off, _PCH2)], idx_d)
            for b in range(_NBUF - 1):
                g_start(b, b)

            def group(g, carry2):
                for b in range(_NBUF):
                    j = g * _NBUF + b
                    g_wait(j, b)
                    s_start(j, b)
                    bn = (b - 1) % _NBUF
                    jn = j + _NBUF - 1

                    @pl.when(j == 0)
                    def _():
                        g_start(jn, bn)

                    @pl.when(jnp.logical_and(j >= 1, jn < _PCH2))
                    def _():
                        s_wait(j - 1, bn)
                        g_start(jn, bn)

                return carry2

            lax.fori_loop(0, ngrp, group, 0)
            for m in range(_PCH2 - _NBUF, _PCH2):
                s_wait(m, m % _NBUF)
            return carry

        lax.fori_loop(0, count // _PCH2, phase, 0)

    @pl.when(cid == 0)
    def _():
        run_core(_C0, 0)

    @pl.when(cid == 1)
    def _():
        run_core(_C1, 16 * _C0)

    plsc.subcore_barrier()
    pltpu.sync_copy(acc.at[pl.ds(sid * _RPS, _RPS)],
                    out_hbm.at[cid, pl.ds(sid * _RPS, _RPS)])


_route = pl.kernel(
    _route_body,
    out_type=jax.ShapeDtypeStruct((_NC, _AROWS, _D), jnp.float32),
    mesh=_mesh,
    scratch_types=[
        pltpu.VMEM((_PCH2, _K), jnp.int32),
        pltpu.VMEM((_PCH2, _K), jnp.int32),
        pltpu.VMEM((_NBUF, _K, _D), jnp.float32),
        pltpu.VMEM_SHARED((_AROWS, _D), jnp.float32),
        pltpu.SemaphoreType.DMA((_NBUF,)),
        pltpu.SemaphoreType.DMA((_NBUF,)),
    ],
)


def _rsqrt_deg(d0_ref, d1_ref):
    deg = d0_ref[:, 0:1] + d1_ref[:, 0:1]
    return lax.rsqrt(jnp.maximum(deg, 1.0))


def _scale_body(x_ref, d0_ref, d1_ref, o_ref):
    o_ref[...] = x_ref[...] * _rsqrt_deg(d0_ref, d1_ref)


_scale = pl.pallas_call(
    _scale_body,
    grid=(_NBLK,),
    in_specs=[
        pl.BlockSpec((_BLK, _D), lambda i: (i, 0)),
        pl.BlockSpec((_BLK, _D), lambda i: (i, 0)),
        pl.BlockSpec((_BLK, _D), lambda i: (i, 0)),
    ],
    out_specs=pl.BlockSpec((_BLK, _D), lambda i: (i, 0)),
    out_shape=jax.ShapeDtypeStruct((_AROWS, _D), jnp.float32),
)


def _mm_stats_body(a0_ref, a1_ref, d0_ref, d1_ref, w_ref, b_ref,
                   h_ref, st_ref):
    i = pl.program_id(0)
    agg = (a0_ref[...] + a1_ref[...]) * _rsqrt_deg(d0_ref, d1_ref)
    z = jnp.dot(agg, w_ref[...], preferred_element_type=jnp.float32,
                precision=lax.Precision.HIGHEST) + b_ref[...]
    h = jnp.maximum(z, 0.0)
    h_ref[...] = h
    rows = i * _BLK + lax.broadcasted_iota(jnp.int32, (_BLK, 1), 0)
    hm = jnp.where(rows < _N, h, 0.0)
    blk = jnp.concatenate(
        [jnp.sum(hm, axis=0, keepdims=True),
         jnp.sum(hm * hm, axis=0, keepdims=True),
         jnp.zeros((6, _D), jnp.float32)], axis=0)

    @pl.when(i == 0)
    def _():
        st_ref[...] = blk

    @pl.when(i > 0)
    def _():
        st_ref[...] += blk


_mm_stats = pl.pallas_call(
    _mm_stats_body,
    grid=(_NBLK,),
    in_specs=[
        pl.BlockSpec((_BLK, _D), lambda i: (i, 0)),
        pl.BlockSpec((_BLK, _D), lambda i: (i, 0)),
        pl.BlockSpec((_BLK, _D), lambda i: (i, 0)),
        pl.BlockSpec((_BLK, _D), lambda i: (i, 0)),
        pl.BlockSpec((_D, _H), lambda i: (0, 0)),
        pl.BlockSpec((1, _H), lambda i: (0, 0)),
    ],
    out_specs=[
        pl.BlockSpec((_BLK, _H), lambda i: (i, 0)),
        pl.BlockSpec((8, _H), lambda i: (0, 0)),
    ],
    out_shape=[
        jax.ShapeDtypeStruct((_AROWS, _H), jnp.float32),
        jax.ShapeDtypeStruct((8, _H), jnp.float32),
    ],
)


def _bnscale_body(h_ref, st_ref, g_ref, be_ref, d0_ref, d1_ref, o_ref):
    i = pl.program_id(0)
    mu = st_ref[0:1, :] * (1.0 / _N)
    var = st_ref[1:2, :] * (1.0 / _N) - mu * mu
    a = g_ref[...] * lax.rsqrt(var + _EPS)
    hn = (h_ref[...] - mu) * a + be_ref[...]
    rows = i * _BLK + lax.broadcasted_iota(jnp.int32, (_BLK, 1), 0)
    o_ref[...] = jnp.where(rows < _N, hn * _rsqrt_deg(d0_ref, d1_ref), 0.0)


_bnscale = pl.pallas_call(
    _bnscale_body,
    grid=(_NBLK,),
    in_specs=[
        pl.BlockSpec((_BLK, _H), lambda i: (i, 0)),
        pl.BlockSpec((8, _H), lambda i: (0, 0)),
        pl.BlockSpec((1, _H), lambda i: (0, 0)),
        pl.BlockSpec((1, _H), lambda i: (0, 0)),
        pl.BlockSpec((_BLK, _D), lambda i: (i, 0)),
        pl.BlockSpec((_BLK, _D), lambda i: (i, 0)),
    ],
    out_specs=pl.BlockSpec((_BLK, _H), lambda i: (i, 0)),
    out_shape=jax.ShapeDtypeStruct((_AROWS, _H), jnp.float32),
)


def _mm_pool_body(a0_ref, a1_ref, d0_ref, d1_ref, w_ref, b_ref, bt_ref,
                  st_ref, pooled_ref, cnt_ref):
    i = pl.program_id(0)
    agg = (a0_ref[...] + a1_ref[...]) * _rsqrt_deg(d0_ref, d1_ref)
    z = jnp.dot(agg, w_ref[...], preferred_element_type=jnp.float32,
                precision=lax.Precision.HIGHEST) + b_ref[...]
    h = jnp.maximum(z, 0.0)
    bt = bt_ref[0]
    gid = lax.broadcasted_iota(jnp.int32, (_G, _BLK), 0)
    onehot = (gid == bt).astype(jnp.float32)
    pooled_blk = jnp.dot(onehot, h, preferred_element_type=jnp.float32,
                         precision=lax.Precision.HIGHEST)
    cnt_blk = jnp.dot(onehot, jnp.ones((_BLK, _H), jnp.float32),
                      preferred_element_type=jnp.float32,
                      precision=lax.Precision.HIGHEST)
    hm = jnp.where(jnp.transpose(bt) < _G, h, 0.0)
    blk = jnp.concatenate(
        [jnp.sum(hm, axis=0, keepdims=True),
         jnp.sum(hm * hm, axis=0, keepdims=True),
         jnp.zeros((6, _H), jnp.float32)], axis=0)

    @pl.when(i == 0)
    def _():
        st_ref[...] = blk
        pooled_ref[...] = pooled_blk
        cnt_ref[...] = cnt_blk

    @pl.when(i > 0)
    def _():
        st_ref[...] += blk
        pooled_ref[...] += pooled_blk
        cnt_ref[...] += cnt_blk


_mm_pool = pl.pallas_call(
    _mm_pool_body,
    grid=(_NBLK,),
    in_specs=[
        pl.BlockSpec((_BLK, _H), lambda i: (i, 0)),
        pl.BlockSpec((_BLK, _H), lambda i: (i, 0)),
        pl.BlockSpec((_BLK, _D), lambda i: (i, 0)),
        pl.BlockSpec((_BLK, _D), lambda i: (i, 0)),
        pl.BlockSpec((_H, _H), lambda i: (0, 0)),
        pl.BlockSpec((1, _H), lambda i: (0, 0)),
        pl.BlockSpec((1, 1, _BLK), lambda i: (i, 0, 0)),
    ],
    out_specs=[
        pl.BlockSpec((8, _H), lambda i: (0, 0)),
        pl.BlockSpec((_G, _H), lambda i: (0, 0)),
        pl.BlockSpec((_G, _H), lambda i: (0, 0)),
    ],
    out_shape=[
        jax.ShapeDtypeStruct((8, _H), jnp.float32),
        jax.ShapeDtypeStruct((_G, _H), jnp.float32),
        jax.ShapeDtypeStruct((_G, _H), jnp.float32),
    ],
)


def _head_body(st_ref, pooled_ref, cnt_ref, g2_ref, b2_ref, gf_ref,
               w1_ref, b1_ref, g3_ref, b3_ref, w2_ref, bc_ref,
               logits_ref, emb_ref):
    mu2 = st_ref[0:1, :] * (1.0 / _N)
    var2 = st_ref[1:2, :] * (1.0 / _N) - mu2 * mu2
    a2 = g2_ref[...] * lax.rsqrt(var2 + _EPS)
    cnt = cnt_ref[...]
    pm = pooled_ref[...] / jnp.maximum(cnt, 1.0)
    p = jnp.where(cnt > 0.0, (pm - mu2) * a2 + b2_ref[...], 0.0)
    z = (jnp.dot(p, w1_ref[0:_H, :], preferred_element_type=jnp.float32,
                 precision=lax.Precision.HIGHEST)
         + jnp.dot(gf_ref[...], w1_ref[_H:_H + _GF, :],
                   preferred_element_type=jnp.float32,
                   precision=lax.Precision.HIGHEST)
         + b1_ref[...])
    z = jnp.maximum(z, 0.0)
    mu3 = jnp.mean(z, axis=0, keepdims=True)
    var3 = jnp.mean((z - mu3) * (z - mu3), axis=0, keepdims=True)
    emb = (z - mu3) * lax.rsqrt(var3 + _EPS) * g3_ref[...] + b3_ref[...]
    emb_ref[...] = emb
    logits_ref[...] = jnp.dot(emb, w2_ref[...],
                              preferred_element_type=jnp.float32,
                              precision=lax.Precision.HIGHEST) + bc_ref[...]


_head = pl.pallas_call(
    _head_body,
    grid=(1,),
    in_specs=[
        pl.BlockSpec((8, _H), lambda i: (0, 0)),
        pl.BlockSpec((_G, _H), lambda i: (0, 0)),
        pl.BlockSpec((_G, _H), lambda i: (0, 0)),
        pl.BlockSpec((1, _H), lambda i: (0, 0)),
        pl.BlockSpec((1, _H), lambda i: (0, 0)),
        pl.BlockSpec((_G, _GF), lambda i: (0, 0)),
        pl.BlockSpec((_H + _GF, _H), lambda i: (0, 0)),
        pl.BlockSpec((1, _H), lambda i: (0, 0)),
        pl.BlockSpec((1, _H), lambda i: (0, 0)),
        pl.BlockSpec((1, _H), lambda i: (0, 0)),
        pl.BlockSpec((_H, _H), lambda i: (0, 0)),
        pl.BlockSpec((1, _H), lambda i: (0, 0)),
    ],
    out_specs=[
        pl.BlockSpec((_G, _H), lambda i: (0, 0)),
        pl.BlockSpec((_G, _H), lambda i: (0, 0)),
    ],
    out_shape=[
        jax.ShapeDtypeStruct((_G, _H), jnp.float32),
        jax.ShapeDtypeStruct((_G, _H), jnp.float32),
    ],
)


def kernel(x, edge_index, batch, graph_features, W1, b1, gamma1, beta1,
           W2, b2, gamma2, beta2, fc1_W, fc1_b, gamma3, beta3, fc2_W, fc2_b):
    f32 = jnp.float32
    src = edge_index[0]
    dst = edge_index[1]
    idx_pad = jnp.full((_EPAD - _E,), _N, jnp.int32)
    srcp = jnp.concatenate([src, idx_pad]).reshape(_NW, _NCHUNK, _K)
    dstp = jnp.concatenate([dst, idx_pad]).reshape(_NW, _NCHUNK, _K)
    src_flat = srcp.reshape(-1, _K)
    dst_flat = dstp.reshape(-1, _K)
    x_pad = jnp.zeros((_AROWS, _D), f32).at[:_N].set(x)
    batch_pad = jnp.full((_AROWS,), _G, jnp.int32).at[:_N].set(batch)
    batch_r = batch_pad.reshape(_NBLK, 1, _BLK)

    ones_row = jnp.ones((_K, _D), f32)
    zeros_rows = jnp.zeros((_RPS, _D), f32)

    row = lambda v: v.reshape(1, -1)

    deg_out = _deg(dstp, ones_row, zeros_rows)
    d0, d1 = deg_out[0], deg_out[1]

    x1 = _scale(x_pad, d0, d1)
    agg1 = _route(x1, src_flat, dst_flat, zeros_rows)
    hrelu, st1 = _mm_stats(agg1[0], agg1[1], d0, d1, W1, row(b1))
    h1s = _bnscale(hrelu, st1, row(gamma1), row(beta1), d0, d1)
    agg2 = _route(h1s, src_flat, dst_flat, zeros_rows)
    st2, pooled, cnt = _mm_pool(agg2[0], agg2[1], d0, d1, W2, row(b2),
                                batch_r)

    fc2_W_pad = jnp.zeros((_H, _H), f32).at[:, :_C].set(fc2_W)
    fc2_b_pad = jnp.zeros((1, _H), f32).at[0, :_C].set(fc2_b)
    logits_pad, embeddings = _head(
        st2, pooled, cnt, row(gamma2), row(beta2), graph_features,
        fc1_W, row(fc1_b), row(gamma3), row(beta3), fc2_W_pad, fc2_b_pad)
    return (logits_pad[:, :_C], embeddings)


# speedup vs baseline: 1.0058x; 1.0058x over previous
import jax
import jax.numpy as jnp
from jax import lax
from jax.experimental import pallas as pl
from jax.experimental.pallas import tpu as pltpu
from jax.experimental.pallas import tpu_sc as plsc

_N = 10000
_E = 320000
_D = 128
_H = 128
_C = 10
_GF = 32
_G = 64
_EPS = 1e-5

_NC = 2
_NS = 16
_NW = _NC * _NS
_K = 64
_NCHUNK = 160
_PCH = 40
_NPH = 4
_EPAD = _NW * _NCHUNK * _K
_AROWS = 10240
_RPS = _AROWS // _NS
_BLK = 256
_NBLK = _AROWS // _BLK

_mesh = plsc.VectorSubcoreMesh(core_axis_name="c", subcore_axis_name="s")


def _deg_body(dst_hbm, ones_hbm, zeros_hbm, out_hbm, idx_d, ones_v, acc,
              sem_s):
    cid = lax.axis_index("c")
    sid = lax.axis_index("s")
    wid = sid * _NC + cid
    pltpu.sync_copy(zeros_hbm, acc.at[pl.ds(sid * _RPS, _RPS)])
    pltpu.sync_copy(ones_hbm, ones_v)
    pltpu.sync_copy(dst_hbm.at[wid], idx_d)
    plsc.subcore_barrier()

    def s_start(j, b):
        pltpu.async_copy(ones_v, acc.at[idx_d.at[j]], sem_s.at[b], add=True)

    def s_wait(j, b):
        pltpu.make_async_copy(ones_v, acc.at[idx_d.at[j]],
                              sem_s.at[b]).wait()

    def body(j, carry):
        @pl.when(j >= _NBUF)
        def _():
            s_wait(j - _NBUF, 0)
        s_start(j, 0)
        return carry

    lax.fori_loop(0, _NCHUNK, body, 0)
    for m in range(_NCHUNK - _NBUF, _NCHUNK):
        s_wait(m, 0)
    plsc.subcore_barrier()
    pltpu.sync_copy(acc.at[pl.ds(sid * _RPS, _RPS)],
                    out_hbm.at[cid, pl.ds(sid * _RPS, _RPS)])


_deg = pl.kernel(
    _deg_body,
    out_type=jax.ShapeDtypeStruct((_NC, _AROWS, _D), jnp.float32),
    mesh=_mesh,
    scratch_types=[
        pltpu.VMEM((_NCHUNK, _K), jnp.int32),
        pltpu.VMEM((_K, _D), jnp.float32),
        pltpu.VMEM_SHARED((_AROWS, _D), jnp.float32),
        pltpu.SemaphoreType.DMA((1,)),
    ],
)


_NBUF = 4
_PCH2 = 32
_C0 = 256
_C1 = 64


def _route_body(x_hbm, src_hbm, dst_hbm, zeros_hbm, out_hbm,
                idx_s, idx_d, rows, acc, sem_g, sem_s):
    cid = lax.axis_index("c")
    sid = lax.axis_index("s")
    pltpu.sync_copy(zeros_hbm, acc.at[pl.ds(sid * _RPS, _RPS)])
    plsc.subcore_barrier()

    def g_start(j, b):
        pltpu.async_copy(x_hbm.at[idx_s.at[j]], rows.at[b], sem_g.at[b])

    def g_wait(j, b):
        pltpu.make_async_copy(x_hbm.at[idx_s.at[j]], rows.at[b],
                              sem_g.at[b]).wait()

    def s_start(j, b):
        pltpu.async_copy(rows.at[b], acc.at[idx_d.at[j]], sem_s.at[b],
                         add=True)

    def s_wait(j, b):
        pltpu.make_async_copy(rows.at[b], acc.at[idx_d.at[j]],
                              sem_s.at[b]).wait()

    def run_core(count, base):
        start0 = base + sid * count
        ngrp = _PCH2 // _NBUF

        def phase(p, carry):
            off = start0 + p * _PCH2
            pltpu.sync_copy(src_hbm.at[pl.ds(off, _PCH2)], idx_s)
            pltpu.sync_copy(dst_hbm.at[pl.ds(off, _PCH2)], idx_d)
            for b in range(_NBUF - 1):
                g_start(b, b)

            def group(g, carry2):
                for b in range(_NBUF):
                    j = g * _NBUF + b
                    g_wait(j, b)
                    s_start(j, b)
                    bn = (b - 1) % _NBUF
                    jn = j + _NBUF - 1

                    @pl.when(j == 0)
                    def _():
                        g_start(jn, bn)

                    @pl.when(jnp.logical_and(j >= 1, jn < _PCH2))
                    def _():
                        s_wait(j - 1, bn)
                        g_start(jn, bn)

                return carry2

            lax.fori_loop(0, ngrp, group, 0)
            for m in range(_PCH2 - _NBUF, _PCH2):
                s_wait(m, m % _NBUF)
            return carry

        lax.fori_loop(0, count // _PCH2, phase, 0)

    @pl.when(cid == 0)
    def _():
        run_core(_C0, 0)

    @pl.when(cid == 1)
    def _():
        run_core(_C1, 16 * _C0)

    plsc.subcore_barrier()
    pltpu.sync_copy(acc.at[pl.ds(sid * _RPS, _RPS)],
                    out_hbm.at[cid, pl.ds(sid * _RPS, _RPS)])


_route = pl.kernel(
    _route_body,
    out_type=jax.ShapeDtypeStruct((_NC, _AROWS, _D), jnp.float32),
    mesh=_mesh,
    scratch_types=[
        pltpu.VMEM((_PCH2, _K), jnp.int32),
        pltpu.VMEM((_PCH2, _K), jnp.int32),
        pltpu.VMEM((_NBUF, _K, _D), jnp.float32),
        pltpu.VMEM_SHARED((_AROWS, _D), jnp.float32),
        pltpu.SemaphoreType.DMA((_NBUF,)),
        pltpu.SemaphoreType.DMA((_NBUF,)),
    ],
)


def _rsqrt_deg(d0_ref, d1_ref):
    deg = d0_ref[:, 0:1] + d1_ref[:, 0:1]
    return lax.rsqrt(jnp.maximum(deg, 1.0))


def _scale_body(x_ref, d0_ref, d1_ref, o_ref):
    o_ref[...] = x_ref[...] * _rsqrt_deg(d0_ref, d1_ref)


_scale = pl.pallas_call(
    _scale_body,
    grid=(_NBLK,),
    in_specs=[
        pl.BlockSpec((_BLK, _D), lambda i: (i, 0)),
        pl.BlockSpec((_BLK, _D), lambda i: (i, 0)),
        pl.BlockSpec((_BLK, _D), lambda i: (i, 0)),
    ],
    out_specs=pl.BlockSpec((_BLK, _D), lambda i: (i, 0)),
    out_shape=jax.ShapeDtypeStruct((_AROWS, _D), jnp.float32),
)


def _mm_stats_body(a0_ref, a1_ref, d0_ref, d1_ref, w_ref, b_ref,
                   h_ref, st_ref):
    i = pl.program_id(0)
    agg = (a0_ref[...] + a1_ref[...]) * _rsqrt_deg(d0_ref, d1_ref)
    z = jnp.dot(agg, w_ref[...], preferred_element_type=jnp.float32,
                precision=lax.Precision.HIGHEST) + b_ref[...]
    h = jnp.maximum(z, 0.0)
    h_ref[...] = h
    rows = i * _BLK + lax.broadcasted_iota(jnp.int32, (_BLK, 1), 0)
    hm = jnp.where(rows < _N, h, 0.0)
    blk = jnp.concatenate(
        [jnp.sum(hm, axis=0, keepdims=True),
         jnp.sum(hm * hm, axis=0, keepdims=True),
         jnp.zeros((6, _D), jnp.float32)], axis=0)

    @pl.when(i == 0)
    def _():
        st_ref[...] = blk

    @pl.when(i > 0)
    def _():
        st_ref[...] += blk


_mm_stats = pl.pallas_call(
    _mm_stats_body,
    grid=(_NBLK,),
    in_specs=[
        pl.BlockSpec((_BLK, _D), lambda i: (i, 0)),
        pl.BlockSpec((_BLK, _D), lambda i: (i, 0)),
        pl.BlockSpec((_BLK, _D), lambda i: (i, 0)),
        pl.BlockSpec((_BLK, _D), lambda i: (i, 0)),
        pl.BlockSpec((_D, _H), lambda i: (0, 0)),
        pl.BlockSpec((1, _H), lambda i: (0, 0)),
    ],
    out_specs=[
        pl.BlockSpec((_BLK, _H), lambda i: (i, 0)),
        pl.BlockSpec((8, _H), lambda i: (0, 0)),
    ],
    out_shape=[
        jax.ShapeDtypeStruct((_AROWS, _H), jnp.float32),
        jax.ShapeDtypeStruct((8, _H), jnp.float32),
    ],
)


def _bnscale_body(h_ref, st_ref, g_ref, be_ref, d0_ref, d1_ref, o_ref):
    i = pl.program_id(0)
    mu = st_ref[0:1, :] * (1.0 / _N)
    var = st_ref[1:2, :] * (1.0 / _N) - mu * mu
    a = g_ref[...] * lax.rsqrt(var + _EPS)
    hn = (h_ref[...] - mu) * a + be_ref[...]
    rows = i * _BLK + lax.broadcasted_iota(jnp.int32, (_BLK, 1), 0)
    o_ref[...] = jnp.where(rows < _N, hn * _rsqrt_deg(d0_ref, d1_ref), 0.0)


_bnscale = pl.pallas_call(
    _bnscale_body,
    grid=(_NBLK,),
    in_specs=[
        pl.BlockSpec((_BLK, _H), lambda i: (i, 0)),
        pl.BlockSpec((8, _H), lambda i: (0, 0)),
        pl.BlockSpec((1, _H), lambda i: (0, 0)),
        pl.BlockSpec((1, _H), lambda i: (0, 0)),
        pl.BlockSpec((_BLK, _D), lambda i: (i, 0)),
        pl.BlockSpec((_BLK, _D), lambda i: (i, 0)),
    ],
    out_specs=pl.BlockSpec((_BLK, _H), lambda i: (i, 0)),
    out_shape=jax.ShapeDtypeStruct((_AROWS, _H), jnp.float32),
)


def _mm_pool_body(a0_ref, a1_ref, d0_ref, d1_ref, w_ref, b_ref, bt_ref,
                  st_ref, pooled_ref, cnt_ref):
    i = pl.program_id(0)
    agg = (a0_ref[...] + a1_ref[...]) * _rsqrt_deg(d0_ref, d1_ref)
    z = jnp.dot(agg, w_ref[...], preferred_element_type=jnp.float32,
                precision=lax.Precision.HIGHEST) + b_ref[...]
    h = jnp.maximum(z, 0.0)
    bt = bt_ref[0]
    gid = lax.broadcasted_iota(jnp.int32, (_G, _BLK), 0)
    onehot = (gid == bt).astype(jnp.float32)
    pooled_blk = jnp.dot(onehot, h, preferred_element_type=jnp.float32,
                         precision=lax.Precision.HIGHEST)
    cnt_blk = jnp.dot(onehot, jnp.ones((_BLK, _H), jnp.float32),
                      preferred_element_type=jnp.float32,
                      precision=lax.Precision.HIGHEST)
    hm = jnp.where(jnp.transpose(bt) < _G, h, 0.0)
    blk = jnp.concatenate(
        [jnp.sum(hm, axis=0, keepdims=True),
         jnp.sum(hm * hm, axis=0, keepdims=True),
         jnp.zeros((6, _H), jnp.float32)], axis=0)

    @pl.when(i == 0)
    def _():
        st_ref[...] = blk
        pooled_ref[...] = pooled_blk
        cnt_ref[...] = cnt_blk

    @pl.when(i > 0)
    def _():
        st_ref[...] += blk
        pooled_ref[...] += pooled_blk
        cnt_ref[...] += cnt_blk


_mm_pool = pl.pallas_call(
    _mm_pool_body,
    grid=(_NBLK,),
    in_specs=[
        pl.BlockSpec((_BLK, _H), lambda i: (i, 0)),
        pl.BlockSpec((_BLK, _H), lambda i: (i, 0)),
        pl.BlockSpec((_BLK, _D), lambda i: (i, 0)),
        pl.BlockSpec((_BLK, _D), lambda i: (i, 0)),
        pl.BlockSpec((_H, _H), lambda i: (0, 0)),
        pl.BlockSpec((1, _H), lambda i: (0, 0)),
        pl.BlockSpec((1, 1, _BLK), lambda i: (i, 0, 0)),
    ],
    out_specs=[
        pl.BlockSpec((8, _H), lambda i: (0, 0)),
        pl.BlockSpec((_G, _H), lambda i: (0, 0)),
        pl.BlockSpec((_G, _H), lambda i: (0, 0)),
    ],
    out_shape=[
        jax.ShapeDtypeStruct((8, _H), jnp.float32),
        jax.ShapeDtypeStruct((_G, _H), jnp.float32),
        jax.ShapeDtypeStruct((_G, _H), jnp.float32),
    ],
)


def _head_body(st_ref, pooled_ref, cnt_ref, g2_ref, b2_ref, gf_ref,
               w1_ref, b1_ref, g3_ref, b3_ref, w2_ref, bc_ref,
               logits_ref, emb_ref):
    mu2 = st_ref[0:1, :] * (1.0 / _N)
    var2 = st_ref[1:2, :] * (1.0 / _N) - mu2 * mu2
    a2 = g2_ref[...] * lax.rsqrt(var2 + _EPS)
    cnt = cnt_ref[...]
    pm = pooled_ref[...] / jnp.maximum(cnt, 1.0)
    p = jnp.where(cnt > 0.0, (pm - mu2) * a2 + b2_ref[...], 0.0)
    z = (jnp.dot(p, w1_ref[0:_H, :], preferred_element_type=jnp.float32,
                 precision=lax.Precision.HIGHEST)
         + jnp.dot(gf_ref[...], w1_ref[_H:_H + _GF, :],
                   preferred_element_type=jnp.float32,
                   precision=lax.Precision.HIGHEST)
         + b1_ref[...])
    z = jnp.maximum(z, 0.0)
    mu3 = jnp.mean(z, axis=0, keepdims=True)
    var3 = jnp.mean((z - mu3) * (z - mu3), axis=0, keepdims=True)
    emb = (z - mu3) * lax.rsqrt(var3 + _EPS) * g3_ref[...] + b3_ref[...]
    emb_ref[...] = emb
    logits_ref[...] = jnp.dot(emb, w2_ref[...],
                              preferred_element_type=jnp.float32,
                              precision=lax.Precision.HIGHEST) + bc_ref[...]


_head = pl.pallas_call(
    _head_body,
    grid=(1,),
    in_specs=[
        pl.BlockSpec((8, _H), lambda i: (0, 0)),
        pl.BlockSpec((_G, _H), lambda i: (0, 0)),
        pl.BlockSpec((_G, _H), lambda i: (0, 0)),
        pl.BlockSpec((1, _H), lambda i: (0, 0)),
        pl.BlockSpec((1, _H), lambda i: (0, 0)),
        pl.BlockSpec((_G, _GF), lambda i: (0, 0)),
        pl.BlockSpec((_H + _GF, _H), lambda i: (0, 0)),
        pl.BlockSpec((1, _H), lambda i: (0, 0)),
        pl.BlockSpec((1, _H), lambda i: (0, 0)),
        pl.BlockSpec((1, _H), lambda i: (0, 0)),
        pl.BlockSpec((_H, _H), lambda i: (0, 0)),
        pl.BlockSpec((1, _H), lambda i: (0, 0)),
    ],
    out_specs=[
        pl.BlockSpec((_G, _H), lambda i: (0, 0)),
        pl.BlockSpec((_G, _H), lambda i: (0, 0)),
    ],
    out_shape=[
        jax.ShapeDtypeStruct((_G, _H), jnp.float32),
        jax.ShapeDtypeStruct((_G, _H), jnp.float32),
    ],
)


def kernel(x, edge_index, batch, graph_features, W1, b1, gamma1, beta1,
           W2, b2, gamma2, beta2, fc1_W, fc1_b, gamma3, beta3, fc2_W, fc2_b):
    f32 = jnp.float32
    src = edge_index[0]
    dst = edge_index[1]
    idx_pad = jnp.full((_EPAD - _E,), _N, jnp.int32)
    srcp = jnp.concatenate([src, idx_pad]).reshape(_NW, _NCHUNK, _K)
    dstp = jnp.concatenate([dst, idx_pad]).reshape(_NW, _NCHUNK, _K)
    src_flat = srcp.reshape(-1, _K)
    dst_flat = dstp.reshape(-1, _K)
    x_pad = jnp.zeros((_AROWS, _D), f32).at[:_N].set(x)
    batch_pad = jnp.full((_AROWS,), _G, jnp.int32).at[:_N].set(batch)
    batch_r = batch_pad.reshape(_NBLK, 1, _BLK)

    ones_row = jnp.ones((_K, _D), f32)
    zeros_rows = jnp.zeros((_RPS, _D), f32)

    row = lambda v: v.reshape(1, -1)

    deg_out = _deg(dstp, ones_row, zeros_rows)
    d0, d1 = deg_out[0], deg_out[1]

    x1 = _scale(x_pad, d0, d1)
    agg1 = _route(x1, src_flat, dst_flat, zeros_rows)
    hrelu, st1 = _mm_stats(agg1[0], agg1[1], d0, d1, W1, row(b1))
    h1s = _bnscale(hrelu, st1, row(gamma1), row(beta1), d0, d1)
    agg2 = _route(h1s, src_flat, dst_flat, zeros_rows)
    st2, pooled, cnt = _mm_pool(agg2[0], agg2[1], d0, d1, W2, row(b2),
                                batch_r)

    fc2_W_pad = jnp.zeros((_H, _H), f32).at[:, :_C].set(fc2_W)
    fc2_b_pad = jnp.zeros((1, _H), f32).at[0, :_C].set(fc2_b)
    logits_pad, embeddings = _head(
        st2, pooled, cnt, row(gamma2), row(beta2), graph_features,
        fc1_W, row(fc1_b), row(gamma3), row(beta3), fc2_W_pad, fc2_b_pad)
    return (logits_pad[:, :_C], embeddings)
